# baseline (speedup 1.0000x reference)
import functools

import numpy as np
import jax
import jax.numpy as jnp
from jax import lax
from jax.experimental import pallas as pl
from jax.experimental.pallas import tpu as pltpu
from jax.experimental.pallas import tpu_sc as plsc

SEQ = 200
DIM = 64
LANES = 16
NC = 2
NS = 16
NW = NC * NS

CHUNK = 256
SUBS = [(0, 128), (128, 128)]


def _pos_table_np() -> np.ndarray:
    pos = np.arange(SEQ, dtype=np.float64)[:, None]
    emb = np.arange(DIM, dtype=np.float64)[None, :]
    tmp = pos / (10000.0 ** (2.0 * emb / DIM))
    even_len = DIM // 2 + DIM % 2
    odd_len = DIM // 2
    out = np.zeros((SEQ, DIM), dtype=np.float64)
    out[:, 0::2] = np.sin(tmp)[:, :even_len]
    out[:, 1::2] = np.cos(tmp)[:, :odd_len]
    return out.astype(np.float32)


_POS = _pos_table_np()


@functools.partial(jax.jit, static_argnames=("total_rows",))
def _lookup(table2, idx, pos, *, total_rows):
    assert total_rows % (NW * CHUNK) == 0
    bpw = total_rows // NW
    nchunk = bpw // CHUNK
    assert nchunk >= 6 and nchunk % 2 == 0
    ocpc = CHUNK // 2

    mesh = plsc.VectorSubcoreMesh(core_axis_name="c", subcore_axis_name="s")

    @functools.partial(
        pl.kernel,
        mesh=mesh,
        out_type=jax.ShapeDtypeStruct((total_rows // 2, 128), jnp.float32),
        compiler_params=pltpu.CompilerParams(needs_layout_passes=False),
        scratch_types=[
            pltpu.VMEM((CHUNK,), jnp.int32),
            pltpu.VMEM((CHUNK,), jnp.int32),
            pltpu.VMEM((CHUNK,), jnp.int32),
            pltpu.VMEM((CHUNK,), jnp.int32),
            pltpu.VMEM((SEQ * DIM,), jnp.float32),
            pltpu.VMEM((CHUNK, 128), jnp.float32),
            pltpu.VMEM((CHUNK, 128), jnp.float32),
            pltpu.VMEM((ocpc, 128), jnp.float32),
            pltpu.VMEM((ocpc, 128), jnp.float32),
            pltpu.SemaphoreType.DMA,
            pltpu.SemaphoreType.DMA,
            pltpu.SemaphoreType.DMA,
        ],
    )
    def body(table_hbm, idx_hbm, pos_hbm, out_hbm,
             raw0, raw1, pidx0, pidx1, pos_v, pair0, pair1, ob0, ob1,
             sem_g, sem_ix, sem_s):
        wid = lax.axis_index("s") * NC + lax.axis_index("c")
        base = wid * bpw
        obase = wid * (bpw // 2)
        raw_b = (raw0, raw1)
        pidx_b = (pidx0, pidx1)
        pair_b = (pair0, pair1)
        out_b = (ob0, ob1)

        def stage_copy(m, b):
            return pltpu.make_async_copy(
                idx_hbm.at[pl.ds(base + m * CHUNK, CHUNK)], raw_b[b], sem_ix)

        def gather_copies(b):
            return [
                pltpu.make_async_copy(
                    table_hbm.at[pidx_b[b].at[pl.ds(o, n)]],
                    pair_b[b].at[pl.ds(o, n)], sem_g)
                for (o, n) in SUBS
            ]

        def scatter_copy(m, b):
            return pltpu.make_async_copy(
                out_b[b], out_hbm.at[pl.ds(obase + m * ocpc, ocpc)], sem_s)

        def make_pidx(b):
            raw, pidx = raw_b[b], pidx_b[b]

            @plsc.parallel_loop(0, CHUNK // LANES, step=1, unroll=4)
            def _sbody(v):
                s = v * LANES
                pidx[pl.ds(s, LANES)] = (
                    lax.shift_right_logical(raw[pl.ds(s, LANES)], 1))

        def process(i, b):
            rowbase = i * CHUNK
            pair, outb, raw = pair_b[b], out_b[b], raw_b[b]

            def gbody(g, carry):
                r0 = g * LANES
                rvec = lax.iota(jnp.int32, LANES) + r0
                rawv = raw[pl.ds(r0, LANES)]
                colbase = (rawv & 1) * DIM
                ocolbase = (rvec & 1) * DIM
                orow = lax.shift_right_logical(rvec, 1)
                posr = lax.rem(rowbase + r0 + lax.iota(jnp.int32, LANES),
                               SEQ)
                poff = posr * DIM

                @plsc.parallel_loop(0, DIM, step=1, unroll=4)
                def _cbody(c):
                    v = plsc.load_gather(pair, [rvec, colbase + c])
                    pv = plsc.load_gather(pos_v, [poff + c])
                    plsc.store_scatter(outb, [orow, ocolbase + c], v + pv)
                return carry
            lax.fori_loop(0, CHUNK // LANES, gbody, 0)

        def one(i, b, wait_s, nxt, nxt2):
            for cp in gather_copies(b):
                cp.wait()
            if nxt:
                stage_copy(i + 1, b ^ 1).wait()
                make_pidx(b ^ 1)
                for cp in gather_copies(b ^ 1):
                    cp.start()
            if wait_s:
                scatter_copy(i, b).wait()
            process(i, b)
            if nxt2:
                stage_copy(i + 2, b).start()
            scatter_copy(i, b).start()

        pltpu.sync_copy(pos_hbm, pos_v)
        pltpu.sync_copy(idx_hbm.at[pl.ds(base, CHUNK)], raw0)
        make_pidx(0)
        for cp in gather_copies(0):
            cp.start()
        stage_copy(1, 1).start()

        one(0, 0, False, True, True)
        one(1, 1, False, True, True)

        def mid(g, carry):
            i0 = 2 * g
            one(i0, 0, True, True, True)
            one(i0 + 1, 1, True, True, True)
            return carry
        lax.fori_loop(1, nchunk // 2 - 1, mid, 0)

        one(nchunk - 2, 0, True, True, False)
        one(nchunk - 1, 1, True, False, False)

        scatter_copy(nchunk - 2, 0).wait()
        scatter_copy(nchunk - 1, 1).wait()

    return body(table2, idx, pos)


def kernel(x, embeddings):
    b, s = x.shape
    idx = x.reshape(-1).astype(jnp.int32)
    table2 = embeddings.reshape(-1, 128)
    pos = jnp.asarray(_POS.reshape(-1))
    out2 = _lookup(table2, idx, pos, total_rows=b * s)
    return out2.reshape(b, s, DIM)


# speedup vs baseline: 2.2474x; 2.2474x over previous
import functools

import numpy as np
import jax
import jax.numpy as jnp
from jax import lax
from jax.experimental import pallas as pl
from jax.experimental.pallas import tpu as pltpu
from jax.experimental.pallas import tpu_sc as plsc

SEQ = 200
DIM = 64
LANES = 16
NC = 2
NS = 16
NW = NC * NS

CHUNK = 400
NBUF = 3
SUBS = [(o, min(128, CHUNK - o)) for o in range(0, CHUNK, 128)]


def _pos_table_np() -> np.ndarray:
    pos = np.arange(SEQ, dtype=np.float64)[:, None]
    emb = np.arange(DIM, dtype=np.float64)[None, :]
    tmp = pos / (10000.0 ** (2.0 * emb / DIM))
    even_len = DIM // 2 + DIM % 2
    odd_len = DIM // 2
    out = np.zeros((SEQ, DIM), dtype=np.float64)
    out[:, 0::2] = np.sin(tmp)[:, :even_len]
    out[:, 1::2] = np.cos(tmp)[:, :odd_len]
    return out.astype(np.float32)


_POS = _pos_table_np()


@functools.partial(jax.jit, static_argnames=("total_rows",))
def _lookup(table, idx, pos, *, total_rows):
    assert total_rows % (NW * CHUNK) == 0
    bpw = total_rows // NW
    nchunk = bpw // CHUNK
    assert (nchunk - 4) % 6 == 0

    mesh = plsc.VectorSubcoreMesh(core_axis_name="c", subcore_axis_name="s")

    @functools.partial(
        pl.kernel,
        mesh=mesh,
        out_type=jax.ShapeDtypeStruct((total_rows, DIM), jnp.float32),
        compiler_params=pltpu.CompilerParams(
            use_tc_tiling_on_sc=False, skip_device_barrier=True),
        scratch_types=[
            pltpu.VMEM((CHUNK,), jnp.int32),
            pltpu.VMEM((CHUNK,), jnp.int32),
            pltpu.VMEM((SEQ, DIM), jnp.float32),
            pltpu.VMEM((CHUNK, DIM), jnp.float32),
            pltpu.VMEM((CHUNK, DIM), jnp.float32),
            pltpu.VMEM((CHUNK, DIM), jnp.float32),
            pltpu.SemaphoreType.DMA,
            pltpu.SemaphoreType.DMA,
            pltpu.SemaphoreType.DMA,
        ],
    )
    def body(table_hbm, idx_hbm, pos_hbm, out_hbm,
             idx_v0, idx_v1, pos_v, r0, r1, r2,
             sem_g, sem_ix, sem_s):
        wid = lax.axis_index("s") * NC + lax.axis_index("c")
        base = wid * bpw
        idx_b = (idx_v0, idx_v1)
        rows_b = (r0, r1, r2)

        def idx_copy(m, ib):
            return pltpu.make_async_copy(
                idx_hbm.at[pl.ds(base + m * CHUNK, CHUNK)], idx_b[ib], sem_ix)

        def gather_copies(ib, rb):
            return [
                pltpu.make_async_copy(
                    table_hbm.at[idx_b[ib].at[pl.ds(o, n)]],
                    rows_b[rb].at[pl.ds(o, n)], sem_g)
                for (o, n) in SUBS
            ]

        def scatter_copy(m, rb):
            return pltpu.make_async_copy(
                rows_b[rb], out_hbm.at[pl.ds(base + m * CHUNK, CHUNK)], sem_s)

        def add_pos(rb):
            rows = rows_b[rb]

            @plsc.parallel_loop(0, SEQ, step=1, unroll=4)
            def _pbody(p):
                for j in range(DIM // LANES):
                    pv = pos_v[p, pl.ds(j * LANES, LANES)]
                    for c in range(CHUNK // SEQ):
                        r = c * SEQ + p
                        plsc.addupdate(
                            rows.at[r, pl.ds(j * LANES, LANES)], pv)

        def one(i, rb, ib, wait_s, next_g, next_ix):
            for cp in gather_copies(ib, rb):
                cp.wait()
            if wait_s:
                scatter_copy(i, (rb + 1) % NBUF).wait()
            if next_g:
                idx_copy(i + 1, ib ^ 1).wait()
                for cp in gather_copies(ib ^ 1, (rb + 1) % NBUF):
                    cp.start()
            if next_ix:
                idx_copy(i + 2, ib).start()
            add_pos(rb)
            scatter_copy(i, rb).start()

        pltpu.sync_copy(pos_hbm, pos_v)
        pltpu.sync_copy(idx_hbm.at[pl.ds(base, CHUNK)], idx_v0)
        for cp in gather_copies(0, 0):
            cp.start()
        idx_copy(1, 1).start()

        one(0, 0, 0, False, True, True)
        one(1, 1, 1, False, True, True)

        def mid(g, carry):
            i0 = 2 + 6 * g
            for k in range(6):
                one(i0 + k, (2 + k) % 3, k % 2, True, True, True)
            return carry
        lax.fori_loop(0, (nchunk - 4) // 6, mid, 0)

        one(nchunk - 2, (nchunk - 2) % 3, (nchunk - 2) % 2, True, True, False)
        one(nchunk - 1, (nchunk - 1) % 3, (nchunk - 1) % 2, True, False, False)

        scatter_copy(nchunk - 2, (nchunk - 2) % 3).wait()
        scatter_copy(nchunk - 1, (nchunk - 1) % 3).wait()

    return body(table, idx, pos)


def kernel(x, embeddings):
    b, s = x.shape
    idx = x.reshape(-1).astype(jnp.int32)
    pos = jnp.asarray(_POS)
    out = _lookup(embeddings, idx, pos, total_rows=b * s)
    return out.reshape(b, s, DIM)
